# Initial kernel scaffold; baseline (speedup 1.0000x reference)
import jax
import jax.numpy as jnp
from jax.experimental import pallas as pl


def kernel(E_user, E_gender, E_item, E_cate, Wa1, ba1, Wa2, ba2, Wa3, ba3, gamma, beta, Wf1, bf1, alpha1, Wf2, bf2, alpha2, Wf3, bf3, user_id, gender, target_item_id, target_cate_id, hist_item_id, hist_cate_id, length):
    raise NotImplementedError("write your pallas kernel here")


# speedup vs baseline: 5.6616x; 5.6616x over previous
import functools

import jax
import jax.numpy as jnp
from jax.experimental import pallas as pl
from jax.experimental.pallas import tpu as pltpu
from jax.experimental.pallas import tpu_sc as plsc


def _sc_gather(E_item, E_cate, E_user, E_gender,
               hist_item_idx, hist_cate_idx, user_idx, gender_idx,
               target_item_idx, target_cate_idx):
  BT = hist_item_idx.shape[0]
  Bn = user_idx.shape[0]
  D = E_item.shape[1] // 2
  DP = E_item.shape[1]
  f32 = jnp.float32
  i32 = jnp.int32
  NC, NS = 2, 16
  NW = NC * NS
  C = 256
  bpw_h = BT // NW
  n_h = bpw_h // C
  bpw_s = Bn // NW
  mesh = plsc.VectorSubcoreMesh(core_axis_name="c", subcore_axis_name="s")

  @functools.partial(
      pl.kernel,
      out_type=(
          jax.ShapeDtypeStruct((BT, DP), f32),
          jax.ShapeDtypeStruct((BT, DP), f32),
          jax.ShapeDtypeStruct((Bn, DP), f32),
          jax.ShapeDtypeStruct((Bn, DP), f32),
          jax.ShapeDtypeStruct((Bn, DP), f32),
          jax.ShapeDtypeStruct((Bn, DP), f32),
      ),
      mesh=mesh,
      scratch_types=[
          pltpu.VMEM((C,), i32),
          pltpu.VMEM((C,), i32),
          pltpu.VMEM((C, DP), f32),
          pltpu.VMEM((C, DP), f32),
          pltpu.VMEM((bpw_s,), i32),
          pltpu.VMEM((bpw_s, DP), f32),
          pltpu.SemaphoreType.DMA,
      ],
  )
  def gather_kernel(ei_hbm, ec_hbm, eu_hbm, eg_hbm,
                    hi_idx_hbm, hc_idx_hbm, u_idx_hbm, g_idx_hbm,
                    ti_idx_hbm, tc_idx_hbm,
                    o_hi, o_hc, o_u, o_g, o_ti, o_tc,
                    idx_i, idx_c, rows_i, rows_c, idx_s, rows_s, sem):
    wid = jax.lax.axis_index("s") * NC + jax.lax.axis_index("c")
    base_h = wid * bpw_h

    @pl.loop(0, n_h)
    def _(ci):
      b = base_h + ci * C
      pltpu.sync_copy(hi_idx_hbm.at[pl.ds(b, C)], idx_i)
      pltpu.sync_copy(hc_idx_hbm.at[pl.ds(b, C)], idx_c)
      pltpu.async_copy(ei_hbm.at[idx_i], rows_i, sem).wait()
      pltpu.async_copy(ec_hbm.at[idx_c], rows_c, sem).wait()
      pltpu.sync_copy(rows_i, o_hi.at[pl.ds(b, C)])
      pltpu.sync_copy(rows_c, o_hc.at[pl.ds(b, C)])

    bs = wid * bpw_s
    for idx_hbm, table, out in ((u_idx_hbm, eu_hbm, o_u),
                                (g_idx_hbm, eg_hbm, o_g),
                                (ti_idx_hbm, ei_hbm, o_ti),
                                (tc_idx_hbm, ec_hbm, o_tc)):
      pltpu.sync_copy(idx_hbm.at[pl.ds(bs, bpw_s)], idx_s)
      pltpu.async_copy(table.at[idx_s], rows_s, sem).wait()
      pltpu.sync_copy(rows_s, out.at[pl.ds(bs, bpw_s)])

  return gather_kernel(E_item, E_cate, E_user, E_gender,
                       hist_item_idx, hist_cate_idx, user_idx, gender_idx,
                       target_item_idx, target_cate_idx)


def _attn_body(hi_ref, hc_ref, u_ref, g_ref, ti_ref, tc_ref, len_ref,
               Wa1_ref, ba1_ref, Wa2_ref, ba2_ref, Wa3_ref, ba3_ref,
               join_ref, *, BB, T, D2):
  f32 = jnp.float32
  D = D2 // 2
  Wa1 = Wa1_ref[...]
  Whd = Wa1[D2:2 * D2, :] - Wa1[2 * D2:3 * D2, :]
  Wm = Wa1[3 * D2:4 * D2, :]
  W1c = jnp.concatenate([Whd, Wm], axis=0)
  Wqd = Wa1[0:D2, :] + Wa1[2 * D2:3 * D2, :]

  q = jnp.concatenate([ti_ref[...][:, :D], tc_ref[...][:, :D]], axis=-1)
  utgc = jnp.concatenate([u_ref[...][:, :D], g_ref[...][:, :D], q], axis=-1)
  h3 = jnp.concatenate([hi_ref[...][:, :D], hc_ref[...][:, :D]],
                       axis=-1).reshape(BB, T, D2)
  X = jnp.concatenate([h3, h3 * q[:, None, :]], axis=-1).reshape(BB * T, 2 * D2)
  Z1 = jnp.dot(X, W1c, preferred_element_type=f32)
  qa = jnp.dot(q, Wqd, preferred_element_type=f32) + ba1_ref[...]
  A1 = jax.nn.sigmoid(Z1.reshape(BB, T, 80) + qa[:, None, :]).reshape(BB * T, 80)
  A2 = jax.nn.sigmoid(
      jnp.dot(A1, Wa2_ref[...], preferred_element_type=f32) + ba2_ref[...])
  s = jnp.dot(A2, Wa3_ref[...], preferred_element_type=f32) + ba3_ref[0, 0]
  s = s.reshape(BB, T) * (1.0 / jnp.sqrt(jnp.float32(D2)))
  pos = jax.lax.broadcasted_iota(jnp.int32, (BB, T), 1)
  s = jnp.where(pos < len_ref[...], s, jnp.float32(-(2.0 ** 32) + 1.0))
  s = s - jnp.max(s, axis=-1, keepdims=True)
  e = jnp.exp(s)
  w = e / jnp.sum(e, axis=-1, keepdims=True)
  rows = [jnp.dot(w[b:b + 1, :], h3[b], preferred_element_type=f32)
          for b in range(BB)]
  attn = jnp.concatenate(rows, axis=0)
  join_ref[...] = jnp.concatenate([utgc, attn], axis=-1)


def _fc_body(join_ref, gamma_ref, beta_ref, Wf1_ref, bf1_ref, a1_ref,
             Wf2_ref, bf2_ref, a2_ref, Wf3_ref, bf3_ref,
             out_ref, logit_ref):
  f32 = jnp.float32
  x = join_ref[...]
  mean = jnp.mean(x, axis=0, keepdims=True)
  var = jnp.mean((x - mean) ** 2, axis=0, keepdims=True)
  xn = (x - mean) / jnp.sqrt(var + 1e-3) * gamma_ref[...] + beta_ref[...]
  h = jnp.dot(xn, Wf1_ref[...], preferred_element_type=f32) + bf1_ref[...]
  h = jnp.maximum(h, 0.0)
  h = h + a1_ref[...] * jnp.minimum(h, 0.0)
  h2 = jnp.dot(h, Wf2_ref[...], preferred_element_type=f32) + bf2_ref[...]
  h2 = jnp.maximum(h2, 0.0)
  h2 = h2 + a2_ref[...] * jnp.minimum(h2, 0.0)
  logit = jnp.dot(h2, Wf3_ref[...], preferred_element_type=f32) + bf3_ref[...]
  m = jnp.max(logit, axis=-1, keepdims=True)
  e = jnp.exp(logit - m)
  out_ref[...] = e / jnp.sum(e, axis=-1, keepdims=True)
  logit_ref[...] = logit


def kernel(E_user, E_gender, E_item, E_cate, Wa1, ba1, Wa2, ba2, Wa3, ba3,
           gamma, beta, Wf1, bf1, alpha1, Wf2, bf2, alpha2, Wf3, bf3,
           user_id, gender, target_item_id, target_cate_id,
           hist_item_id, hist_cate_id, length):
  B, T = hist_item_id.shape
  D = E_item.shape[1]
  D2 = 2 * D
  f32 = jnp.float32
  i32 = jnp.int32

  pad = lambda E: jnp.pad(E, ((0, 0), (0, D)))
  hi, hc, u, g, ti, tc = _sc_gather(
      pad(E_item), pad(E_cate), pad(E_user), pad(E_gender),
      hist_item_id.reshape(B * T).astype(i32),
      hist_cate_id.reshape(B * T).astype(i32),
      user_id.astype(i32), gender.astype(i32),
      target_item_id.astype(i32), target_cate_id.astype(i32))

  BB = 16
  NB = B // BB
  len_i = length.astype(i32).reshape(B, 1)

  join = pl.pallas_call(
      functools.partial(_attn_body, BB=BB, T=T, D2=D2),
      grid=(NB,),
      in_specs=[
          pl.BlockSpec((BB * T, D2), lambda i: (i, 0)),
          pl.BlockSpec((BB * T, D2), lambda i: (i, 0)),
          pl.BlockSpec((BB, D2), lambda i: (i, 0)),
          pl.BlockSpec((BB, D2), lambda i: (i, 0)),
          pl.BlockSpec((BB, D2), lambda i: (i, 0)),
          pl.BlockSpec((BB, D2), lambda i: (i, 0)),
          pl.BlockSpec((BB, 1), lambda i: (i, 0)),
          pl.BlockSpec((4 * D2, 80), lambda i: (0, 0)),
          pl.BlockSpec((1, 80), lambda i: (0, 0)),
          pl.BlockSpec((80, 40), lambda i: (0, 0)),
          pl.BlockSpec((1, 40), lambda i: (0, 0)),
          pl.BlockSpec((40, 1), lambda i: (0, 0)),
          pl.BlockSpec((1, 1), lambda i: (0, 0)),
      ],
      out_specs=pl.BlockSpec((BB, 6 * D), lambda i: (i, 0)),
      out_shape=jax.ShapeDtypeStruct((B, 6 * D), f32),
  )(hi, hc, u, g, ti, tc, len_i, Wa1, ba1.reshape(1, -1), Wa2,
    ba2.reshape(1, -1), Wa3, ba3.reshape(1, 1))

  out, logit = pl.pallas_call(
      _fc_body,
      out_shape=(jax.ShapeDtypeStruct((B, 2), f32),
                 jax.ShapeDtypeStruct((B, 2), f32)),
  )(join, gamma.reshape(1, -1), beta.reshape(1, -1),
    Wf1, bf1.reshape(1, -1), alpha1.reshape(1, -1),
    Wf2, bf2.reshape(1, -1), alpha2.reshape(1, -1),
    Wf3, bf3.reshape(1, -1))
  return out, logit


# speedup vs baseline: 6.3081x; 1.1142x over previous
import functools

import jax
import jax.numpy as jnp
from jax.experimental import pallas as pl
from jax.experimental.pallas import tpu as pltpu
from jax.experimental.pallas import tpu_sc as plsc


def _sc_gather(E_item, E_cate, E_user, E_gender,
               hist_item_idx, hist_cate_idx, user_idx, gender_idx,
               target_item_idx, target_cate_idx):
  BT = hist_item_idx.shape[0]
  Bn = user_idx.shape[0]
  DP = E_item.shape[1]
  dt = E_item.dtype
  i32 = jnp.int32
  NC, NS = 2, 16
  NW = NC * NS
  C = 200
  bpw_h = BT // NW
  n_h = bpw_h // C
  bpw_s = Bn // NW
  mesh = plsc.VectorSubcoreMesh(core_axis_name="c", subcore_axis_name="s")

  @functools.partial(
      pl.kernel,
      out_type=(
          jax.ShapeDtypeStruct((BT, DP), dt),
          jax.ShapeDtypeStruct((BT, DP), dt),
          jax.ShapeDtypeStruct((Bn, DP), dt),
          jax.ShapeDtypeStruct((Bn, DP), dt),
          jax.ShapeDtypeStruct((Bn, DP), dt),
          jax.ShapeDtypeStruct((Bn, DP), dt),
      ),
      mesh=mesh,
      scratch_types=[
          pltpu.VMEM((C,), i32),
          pltpu.VMEM((C,), i32),
          pltpu.VMEM((C,), i32),
          pltpu.VMEM((C,), i32),
          pltpu.VMEM((C, DP), dt),
          pltpu.VMEM((C, DP), dt),
          pltpu.VMEM((C, DP), dt),
          pltpu.VMEM((C, DP), dt),
          pltpu.VMEM((bpw_s,), i32),
          pltpu.VMEM((bpw_s, DP), dt),
          pltpu.SemaphoreType.DMA,
          pltpu.SemaphoreType.DMA,
          pltpu.SemaphoreType.DMA,
      ],
  )
  def gather_kernel(ei_hbm, ec_hbm, eu_hbm, eg_hbm,
                    hi_idx_hbm, hc_idx_hbm, u_idx_hbm, g_idx_hbm,
                    ti_idx_hbm, tc_idx_hbm,
                    o_hi, o_hc, o_u, o_g, o_ti, o_tc,
                    ii0, ic0, ii1, ic1, ri0, rc0, ri1, rc1, idx_s, rows_s,
                    sem0, sem1, sem_s):
    wid = jax.lax.axis_index("s") * NC + jax.lax.axis_index("c")
    base_h = wid * bpw_h
    bufs = ((ii0, ic0, ri0, rc0, sem0),
            (ii1, ic1, ri1, rc1, sem1))

    def load_start(ci, s):
      ii, ic, ri, rc, sem = bufs[s]
      b = base_h + ci * C
      pltpu.sync_copy(hi_idx_hbm.at[pl.ds(b, C)], ii)
      pltpu.sync_copy(hc_idx_hbm.at[pl.ds(b, C)], ic)
      pltpu.async_copy(ei_hbm.at[ii], ri, sem)
      pltpu.async_copy(ec_hbm.at[ic], rc, sem)

    def drain_store(ci, s):
      ii, ic, ri, rc, sem = bufs[s]
      b = base_h + ci * C
      pltpu.make_async_copy(ei_hbm.at[ii], ri, sem).wait()
      pltpu.make_async_copy(ec_hbm.at[ic], rc, sem).wait()
      pltpu.sync_copy(ri, o_hi.at[pl.ds(b, C)])
      pltpu.sync_copy(rc, o_hc.at[pl.ds(b, C)])

    load_start(0, 0)
    @pl.loop(0, n_h // 2 - 1)
    def _(j):
      c = 2 * j
      load_start(c + 1, 1)
      drain_store(c, 0)
      load_start(c + 2, 0)
      drain_store(c + 1, 1)
    load_start(n_h - 1, 1)
    drain_store(n_h - 2, 0)
    drain_store(n_h - 1, 1)

    bs = wid * bpw_s
    for idx_hbm, table, out in ((u_idx_hbm, eu_hbm, o_u),
                                (g_idx_hbm, eg_hbm, o_g),
                                (ti_idx_hbm, ei_hbm, o_ti),
                                (tc_idx_hbm, ec_hbm, o_tc)):
      pltpu.sync_copy(idx_hbm.at[pl.ds(bs, bpw_s)], idx_s)
      pltpu.async_copy(table.at[idx_s], rows_s, sem_s).wait()
      pltpu.sync_copy(rows_s, out.at[pl.ds(bs, bpw_s)])

  return gather_kernel(E_item, E_cate, E_user, E_gender,
                       hist_item_idx, hist_cate_idx, user_idx, gender_idx,
                       target_item_idx, target_cate_idx)


def _attn_body(hi_ref, hc_ref, u_ref, g_ref, ti_ref, tc_ref, len_ref,
               Wa1_ref, ba1_ref, Wa2_ref, ba2_ref, Wa3_ref, ba3_ref,
               join_ref, *, BB, T, D2):
  f32 = jnp.float32
  D = D2 // 2
  Wa1 = Wa1_ref[...]
  Whd = Wa1[D2:2 * D2, :] - Wa1[2 * D2:3 * D2, :]
  Wm = Wa1[3 * D2:4 * D2, :]
  W1c = jnp.concatenate([Whd, Wm], axis=0)
  Wqd = Wa1[0:D2, :] + Wa1[2 * D2:3 * D2, :]

  q = jnp.concatenate([ti_ref[...][:, :D], tc_ref[...][:, :D]],
                      axis=-1).astype(f32)
  utgc = jnp.concatenate([u_ref[...][:, :D].astype(f32),
                          g_ref[...][:, :D].astype(f32), q], axis=-1)
  h3 = jnp.concatenate([hi_ref[...][:, :D], hc_ref[...][:, :D]],
                       axis=-1).astype(f32).reshape(BB, T, D2)
  X = jnp.concatenate([h3, h3 * q[:, None, :]], axis=-1).reshape(BB * T, 2 * D2)
  Z1 = jnp.dot(X, W1c, preferred_element_type=f32)
  qa = jnp.dot(q, Wqd, preferred_element_type=f32) + ba1_ref[...]
  A1 = jax.nn.sigmoid(Z1.reshape(BB, T, 80) + qa[:, None, :]).reshape(BB * T, 80)
  A2 = jax.nn.sigmoid(
      jnp.dot(A1, Wa2_ref[...], preferred_element_type=f32) + ba2_ref[...])
  s = jnp.dot(A2, Wa3_ref[...], preferred_element_type=f32) + ba3_ref[0, 0]
  s = s.reshape(BB, T) * (1.0 / jnp.sqrt(jnp.float32(D2)))
  pos = jax.lax.broadcasted_iota(jnp.int32, (BB, T), 1)
  s = jnp.where(pos < len_ref[...], s, jnp.float32(-(2.0 ** 32) + 1.0))
  s = s - jnp.max(s, axis=-1, keepdims=True)
  e = jnp.exp(s)
  w = e / jnp.sum(e, axis=-1, keepdims=True)
  rows = [jnp.dot(w[b:b + 1, :], h3[b], preferred_element_type=f32)
          for b in range(BB)]
  attn = jnp.concatenate(rows, axis=0)
  join_ref[...] = jnp.concatenate([utgc, attn], axis=-1)


def _fc_body(join_ref, gamma_ref, beta_ref, Wf1_ref, bf1_ref, a1_ref,
             Wf2_ref, bf2_ref, a2_ref, Wf3_ref, bf3_ref,
             out_ref, logit_ref):
  f32 = jnp.float32
  x = join_ref[...]
  mean = jnp.mean(x, axis=0, keepdims=True)
  var = jnp.mean((x - mean) ** 2, axis=0, keepdims=True)
  xn = (x - mean) / jnp.sqrt(var + 1e-3) * gamma_ref[...] + beta_ref[...]
  h = jnp.dot(xn, Wf1_ref[...], preferred_element_type=f32) + bf1_ref[...]
  h = jnp.maximum(h, 0.0)
  h = h + a1_ref[...] * jnp.minimum(h, 0.0)
  h2 = jnp.dot(h, Wf2_ref[...], preferred_element_type=f32) + bf2_ref[...]
  h2 = jnp.maximum(h2, 0.0)
  h2 = h2 + a2_ref[...] * jnp.minimum(h2, 0.0)
  logit = jnp.dot(h2, Wf3_ref[...], preferred_element_type=f32) + bf3_ref[...]
  m = jnp.max(logit, axis=-1, keepdims=True)
  e = jnp.exp(logit - m)
  out_ref[...] = e / jnp.sum(e, axis=-1, keepdims=True)
  logit_ref[...] = logit


def kernel(E_user, E_gender, E_item, E_cate, Wa1, ba1, Wa2, ba2, Wa3, ba3,
           gamma, beta, Wf1, bf1, alpha1, Wf2, bf2, alpha2, Wf3, bf3,
           user_id, gender, target_item_id, target_cate_id,
           hist_item_id, hist_cate_id, length):
  B, T = hist_item_id.shape
  D = E_item.shape[1]
  D2 = 2 * D
  f32 = jnp.float32
  i32 = jnp.int32

  pad = lambda E: jnp.pad(E, ((0, 0), (0, D)))
  hi, hc, u, g, ti, tc = _sc_gather(
      pad(E_item), pad(E_cate), pad(E_user), pad(E_gender),
      hist_item_id.reshape(B * T).astype(i32),
      hist_cate_id.reshape(B * T).astype(i32),
      user_id.astype(i32), gender.astype(i32),
      target_item_id.astype(i32), target_cate_id.astype(i32))

  BB = 16
  NB = B // BB
  len_i = length.astype(i32).reshape(B, 1)

  join = pl.pallas_call(
      functools.partial(_attn_body, BB=BB, T=T, D2=D2),
      grid=(NB,),
      in_specs=[
          pl.BlockSpec((BB * T, D2), lambda i: (i, 0)),
          pl.BlockSpec((BB * T, D2), lambda i: (i, 0)),
          pl.BlockSpec((BB, D2), lambda i: (i, 0)),
          pl.BlockSpec((BB, D2), lambda i: (i, 0)),
          pl.BlockSpec((BB, D2), lambda i: (i, 0)),
          pl.BlockSpec((BB, D2), lambda i: (i, 0)),
          pl.BlockSpec((BB, 1), lambda i: (i, 0)),
          pl.BlockSpec((4 * D2, 80), lambda i: (0, 0)),
          pl.BlockSpec((1, 80), lambda i: (0, 0)),
          pl.BlockSpec((80, 40), lambda i: (0, 0)),
          pl.BlockSpec((1, 40), lambda i: (0, 0)),
          pl.BlockSpec((40, 1), lambda i: (0, 0)),
          pl.BlockSpec((1, 1), lambda i: (0, 0)),
      ],
      out_specs=pl.BlockSpec((BB, 6 * D), lambda i: (i, 0)),
      out_shape=jax.ShapeDtypeStruct((B, 6 * D), f32),
  )(hi, hc, u, g, ti, tc, len_i, Wa1, ba1.reshape(1, -1), Wa2,
    ba2.reshape(1, -1), Wa3, ba3.reshape(1, 1))

  out, logit = pl.pallas_call(
      _fc_body,
      out_shape=(jax.ShapeDtypeStruct((B, 2), f32),
                 jax.ShapeDtypeStruct((B, 2), f32)),
  )(join, gamma.reshape(1, -1), beta.reshape(1, -1),
    Wf1, bf1.reshape(1, -1), alpha1.reshape(1, -1),
    Wf2, bf2.reshape(1, -1), alpha2.reshape(1, -1),
    Wf3, bf3.reshape(1, -1))
  return out, logit


# speedup vs baseline: 7.4234x; 1.1768x over previous
import functools

import jax
import jax.numpy as jnp
from jax.experimental import pallas as pl
from jax.experimental.pallas import tpu as pltpu
from jax.experimental.pallas import tpu_sc as plsc


def _sc_gather(E_item, E_cate, E_user, E_gender,
               hist_item_idx, hist_cate_idx, user_idx, gender_idx,
               target_item_idx, target_cate_idx):
  BT = hist_item_idx.shape[0]
  Bn = user_idx.shape[0]
  DP = E_item.shape[1]
  dt = E_item.dtype
  i32 = jnp.int32
  NC, NS = 2, 16
  NW = NC * NS
  C = 200
  bpw_h = BT // NW
  n_h = bpw_h // C
  bpw_s = Bn // NW
  mesh = plsc.VectorSubcoreMesh(core_axis_name="c", subcore_axis_name="s")

  @functools.partial(
      pl.kernel,
      out_type=(
          jax.ShapeDtypeStruct((BT, DP), dt),
          jax.ShapeDtypeStruct((BT, DP), dt),
          jax.ShapeDtypeStruct((Bn, DP), dt),
          jax.ShapeDtypeStruct((Bn, DP), dt),
          jax.ShapeDtypeStruct((Bn, DP), dt),
          jax.ShapeDtypeStruct((Bn, DP), dt),
      ),
      mesh=mesh,
      scratch_types=[
          pltpu.VMEM((C,), i32),
          pltpu.VMEM((C,), i32),
          pltpu.VMEM((C,), i32),
          pltpu.VMEM((C,), i32),
          pltpu.VMEM((C, DP), dt),
          pltpu.VMEM((C, DP), dt),
          pltpu.VMEM((C, DP), dt),
          pltpu.VMEM((C, DP), dt),
          pltpu.VMEM((bpw_s,), i32),
          pltpu.VMEM((bpw_s, DP), dt),
          pltpu.SemaphoreType.DMA,
          pltpu.SemaphoreType.DMA,
          pltpu.SemaphoreType.DMA,
      ],
  )
  def gather_kernel(ei_hbm, ec_hbm, eu_hbm, eg_hbm,
                    hi_idx_hbm, hc_idx_hbm, u_idx_hbm, g_idx_hbm,
                    ti_idx_hbm, tc_idx_hbm,
                    o_hi, o_hc, o_u, o_g, o_ti, o_tc,
                    ii0, ic0, ii1, ic1, ri0, rc0, ri1, rc1, idx_s, rows_s,
                    sem0, sem1, sem_s):
    wid = jax.lax.axis_index("s") * NC + jax.lax.axis_index("c")
    base_h = wid * bpw_h
    bufs = ((ii0, ic0, ri0, rc0, sem0),
            (ii1, ic1, ri1, rc1, sem1))

    def load_start(ci, s):
      ii, ic, ri, rc, sem = bufs[s]
      b = base_h + ci * C
      pltpu.sync_copy(hi_idx_hbm.at[pl.ds(b, C)], ii)
      pltpu.sync_copy(hc_idx_hbm.at[pl.ds(b, C)], ic)
      pltpu.async_copy(ei_hbm.at[ii], ri, sem)
      pltpu.async_copy(ec_hbm.at[ic], rc, sem)

    def drain_store(ci, s):
      ii, ic, ri, rc, sem = bufs[s]
      b = base_h + ci * C
      pltpu.make_async_copy(ei_hbm.at[ii], ri, sem).wait()
      pltpu.make_async_copy(ec_hbm.at[ic], rc, sem).wait()
      pltpu.sync_copy(ri, o_hi.at[pl.ds(b, C)])
      pltpu.sync_copy(rc, o_hc.at[pl.ds(b, C)])

    load_start(0, 0)
    @pl.loop(0, n_h // 2 - 1)
    def _(j):
      c = 2 * j
      load_start(c + 1, 1)
      drain_store(c, 0)
      load_start(c + 2, 0)
      drain_store(c + 1, 1)
    load_start(n_h - 1, 1)
    drain_store(n_h - 2, 0)
    drain_store(n_h - 1, 1)

    bs = wid * bpw_s
    for idx_hbm, table, out in ((u_idx_hbm, eu_hbm, o_u),
                                (g_idx_hbm, eg_hbm, o_g),
                                (ti_idx_hbm, ei_hbm, o_ti),
                                (tc_idx_hbm, ec_hbm, o_tc)):
      pltpu.sync_copy(idx_hbm.at[pl.ds(bs, bpw_s)], idx_s)
      pltpu.async_copy(table.at[idx_s], rows_s, sem_s).wait()
      pltpu.sync_copy(rows_s, out.at[pl.ds(bs, bpw_s)])

  return gather_kernel(E_item, E_cate, E_user, E_gender,
                       hist_item_idx, hist_cate_idx, user_idx, gender_idx,
                       target_item_idx, target_cate_idx)


def _attn_body(hi_ref, hc_ref, u_ref, g_ref, ti_ref, tc_ref, len_ref,
               Wa1_ref, ba1_ref, Wa2_ref, ba2_ref, Wa3_ref, ba3_ref,
               join_ref, *, BB, T, D2):
  f32 = jnp.float32
  D = D2 // 2
  Wa1 = Wa1_ref[...]
  Whd = Wa1[D2:2 * D2, :] - Wa1[2 * D2:3 * D2, :]
  Wm = Wa1[3 * D2:4 * D2, :]
  W1c = jnp.concatenate([Whd, Wm], axis=0)
  Wqd = Wa1[0:D2, :] + Wa1[2 * D2:3 * D2, :]

  q = jnp.concatenate([ti_ref[...][:, :D], tc_ref[...][:, :D]],
                      axis=-1).astype(f32)
  utgc = jnp.concatenate([u_ref[...][:, :D].astype(f32),
                          g_ref[...][:, :D].astype(f32), q], axis=-1)
  h3 = jnp.concatenate([hi_ref[...][:, :D], hc_ref[...][:, :D]],
                       axis=-1).astype(f32).reshape(BB, T, D2)
  X = jnp.concatenate([h3, h3 * q[:, None, :]], axis=-1).reshape(BB * T, 2 * D2)
  Z1 = jnp.dot(X, W1c, preferred_element_type=f32)
  qa = jnp.dot(q, Wqd, preferred_element_type=f32) + ba1_ref[...]
  A1 = jax.nn.sigmoid(Z1.reshape(BB, T, 80) + qa[:, None, :]).reshape(BB * T, 80)
  A2 = jax.nn.sigmoid(
      jnp.dot(A1, Wa2_ref[...], preferred_element_type=f32) + ba2_ref[...])
  s = jnp.dot(A2, Wa3_ref[...], preferred_element_type=f32) + ba3_ref[0, 0]
  s = s.reshape(BB, T) * (1.0 / jnp.sqrt(jnp.float32(D2)))
  pos = jax.lax.broadcasted_iota(jnp.int32, (BB, T), 1)
  s = jnp.where(pos < len_ref[...], s, jnp.float32(-(2.0 ** 32) + 1.0))
  s = s - jnp.max(s, axis=-1, keepdims=True)
  e = jnp.exp(s)
  w = e / jnp.sum(e, axis=-1, keepdims=True)
  rows = [jnp.dot(w[b:b + 1, :], h3[b], preferred_element_type=f32)
          for b in range(BB)]
  attn = jnp.concatenate(rows, axis=0)
  join_ref[...] = jnp.concatenate([utgc, attn], axis=-1)


def _fc_body(join_ref, gamma_ref, beta_ref, Wf1_ref, bf1_ref, a1_ref,
             Wf2_ref, bf2_ref, a2_ref, Wf3_ref, bf3_ref,
             out_ref, logit_ref):
  f32 = jnp.float32
  x = join_ref[...]
  mean = jnp.mean(x, axis=0, keepdims=True)
  var = jnp.mean((x - mean) ** 2, axis=0, keepdims=True)
  xn = (x - mean) / jnp.sqrt(var + 1e-3) * gamma_ref[...] + beta_ref[...]
  h = jnp.dot(xn, Wf1_ref[...], preferred_element_type=f32) + bf1_ref[...]
  h = jnp.maximum(h, 0.0)
  h = h + a1_ref[...] * jnp.minimum(h, 0.0)
  h2 = jnp.dot(h, Wf2_ref[...], preferred_element_type=f32) + bf2_ref[...]
  h2 = jnp.maximum(h2, 0.0)
  h2 = h2 + a2_ref[...] * jnp.minimum(h2, 0.0)
  logit = jnp.dot(h2, Wf3_ref[...], preferred_element_type=f32) + bf3_ref[...]
  m = jnp.max(logit, axis=-1, keepdims=True)
  e = jnp.exp(logit - m)
  out_ref[...] = e / jnp.sum(e, axis=-1, keepdims=True)
  logit_ref[...] = logit


def kernel(E_user, E_gender, E_item, E_cate, Wa1, ba1, Wa2, ba2, Wa3, ba3,
           gamma, beta, Wf1, bf1, alpha1, Wf2, bf2, alpha2, Wf3, bf3,
           user_id, gender, target_item_id, target_cate_id,
           hist_item_id, hist_cate_id, length):
  B, T = hist_item_id.shape
  D = E_item.shape[1]
  D2 = 2 * D
  f32 = jnp.float32
  i32 = jnp.int32

  pad = lambda E: jnp.pad(E, ((0, 0), (0, D)))
  Ei_p, Ec_p, Eu_p, Eg_p = pad(E_item), pad(E_cate), pad(E_user), pad(E_gender)
  hi_idx = hist_item_id.reshape(B * T).astype(i32)
  hc_idx = hist_cate_id.reshape(B * T).astype(i32)
  u_idx = user_id.astype(i32)
  g_idx = gender.astype(i32)
  ti_idx = target_item_id.astype(i32)
  tc_idx = target_cate_id.astype(i32)

  BB = 16
  len_i = length.astype(i32).reshape(B, 1)

  NCH = 4
  Bc = B // NCH
  BTc = Bc * T
  attn_call = pl.pallas_call(
      functools.partial(_attn_body, BB=BB, T=T, D2=D2),
      grid=(Bc // BB,),
      in_specs=[
          pl.BlockSpec((BB * T, D2), lambda i: (i, 0)),
          pl.BlockSpec((BB * T, D2), lambda i: (i, 0)),
          pl.BlockSpec((BB, D2), lambda i: (i, 0)),
          pl.BlockSpec((BB, D2), lambda i: (i, 0)),
          pl.BlockSpec((BB, D2), lambda i: (i, 0)),
          pl.BlockSpec((BB, D2), lambda i: (i, 0)),
          pl.BlockSpec((BB, 1), lambda i: (i, 0)),
          pl.BlockSpec((4 * D2, 80), lambda i: (0, 0)),
          pl.BlockSpec((1, 80), lambda i: (0, 0)),
          pl.BlockSpec((80, 40), lambda i: (0, 0)),
          pl.BlockSpec((1, 40), lambda i: (0, 0)),
          pl.BlockSpec((40, 1), lambda i: (0, 0)),
          pl.BlockSpec((1, 1), lambda i: (0, 0)),
      ],
      out_specs=pl.BlockSpec((BB, 6 * D), lambda i: (i, 0)),
      out_shape=jax.ShapeDtypeStruct((Bc, 6 * D), f32),
  )
  joins = []
  for k in range(NCH):
    st, sb = k * BTc, k * Bc
    hi, hc, u, g, ti, tc = _sc_gather(
        Ei_p, Ec_p, Eu_p, Eg_p,
        hi_idx[st:st + BTc], hc_idx[st:st + BTc],
        u_idx[sb:sb + Bc], g_idx[sb:sb + Bc],
        ti_idx[sb:sb + Bc], tc_idx[sb:sb + Bc])
    joins.append(attn_call(
        hi, hc, u, g, ti, tc, len_i[sb:sb + Bc], Wa1, ba1.reshape(1, -1),
        Wa2, ba2.reshape(1, -1), Wa3, ba3.reshape(1, 1)))
  join = jnp.concatenate(joins, axis=0)

  out, logit = pl.pallas_call(
      _fc_body,
      out_shape=(jax.ShapeDtypeStruct((B, 2), f32),
                 jax.ShapeDtypeStruct((B, 2), f32)),
  )(join, gamma.reshape(1, -1), beta.reshape(1, -1),
    Wf1, bf1.reshape(1, -1), alpha1.reshape(1, -1),
    Wf2, bf2.reshape(1, -1), alpha2.reshape(1, -1),
    Wf3, bf3.reshape(1, -1))
  return out, logit


# speedup vs baseline: 7.5545x; 1.0177x over previous
import functools

import jax
import jax.numpy as jnp
from jax.experimental import pallas as pl
from jax.experimental.pallas import tpu as pltpu
from jax.experimental.pallas import tpu_sc as plsc


def _sc_gather(E_item, E_cate, E_user, E_gender,
               hist_item_idx, hist_cate_idx, user_idx, gender_idx,
               target_item_idx, target_cate_idx):
  BT = hist_item_idx.shape[0]
  Bn = user_idx.shape[0]
  DP = E_item.shape[1]
  dt = E_item.dtype
  i32 = jnp.int32
  NC, NS = 2, 16
  NW = NC * NS
  C = 200
  bpw_h = BT // NW
  n_h = bpw_h // C
  bpw_s = Bn // NW
  mesh = plsc.VectorSubcoreMesh(core_axis_name="c", subcore_axis_name="s")

  @functools.partial(
      pl.kernel,
      out_type=(
          jax.ShapeDtypeStruct((BT, DP), dt),
          jax.ShapeDtypeStruct((BT, DP), dt),
          jax.ShapeDtypeStruct((Bn, DP), dt),
          jax.ShapeDtypeStruct((Bn, DP), dt),
          jax.ShapeDtypeStruct((Bn, DP), dt),
          jax.ShapeDtypeStruct((Bn, DP), dt),
      ),
      mesh=mesh,
      scratch_types=[
          pltpu.VMEM((C,), i32),
          pltpu.VMEM((C,), i32),
          pltpu.VMEM((C,), i32),
          pltpu.VMEM((C,), i32),
          pltpu.VMEM((C, DP), dt),
          pltpu.VMEM((C, DP), dt),
          pltpu.VMEM((C, DP), dt),
          pltpu.VMEM((C, DP), dt),
          pltpu.VMEM((bpw_s,), i32),
          pltpu.VMEM((bpw_s, DP), dt),
          pltpu.SemaphoreType.DMA,
          pltpu.SemaphoreType.DMA,
          pltpu.SemaphoreType.DMA,
      ],
  )
  def gather_kernel(ei_hbm, ec_hbm, eu_hbm, eg_hbm,
                    hi_idx_hbm, hc_idx_hbm, u_idx_hbm, g_idx_hbm,
                    ti_idx_hbm, tc_idx_hbm,
                    o_hi, o_hc, o_u, o_g, o_ti, o_tc,
                    ii0, ic0, ii1, ic1, ri0, rc0, ri1, rc1, idx_s, rows_s,
                    sem0, sem1, sem_s):
    wid = jax.lax.axis_index("s") * NC + jax.lax.axis_index("c")
    base_h = wid * bpw_h
    bufs = ((ii0, ic0, ri0, rc0, sem0),
            (ii1, ic1, ri1, rc1, sem1))

    def load_start(ci, s):
      ii, ic, ri, rc, sem = bufs[s]
      b = base_h + ci * C
      pltpu.sync_copy(hi_idx_hbm.at[pl.ds(b, C)], ii)
      pltpu.sync_copy(hc_idx_hbm.at[pl.ds(b, C)], ic)
      pltpu.async_copy(ei_hbm.at[ii], ri, sem)
      pltpu.async_copy(ec_hbm.at[ic], rc, sem)

    def drain_store(ci, s):
      ii, ic, ri, rc, sem = bufs[s]
      b = base_h + ci * C
      pltpu.make_async_copy(ei_hbm.at[ii], ri, sem).wait()
      pltpu.make_async_copy(ec_hbm.at[ic], rc, sem).wait()
      pltpu.sync_copy(ri, o_hi.at[pl.ds(b, C)])
      pltpu.sync_copy(rc, o_hc.at[pl.ds(b, C)])

    load_start(0, 0)
    @pl.loop(0, n_h // 2 - 1)
    def _(j):
      c = 2 * j
      load_start(c + 1, 1)
      drain_store(c, 0)
      load_start(c + 2, 0)
      drain_store(c + 1, 1)
    load_start(n_h - 1, 1)
    drain_store(n_h - 2, 0)
    drain_store(n_h - 1, 1)

    bs = wid * bpw_s
    for idx_hbm, table, out in ((u_idx_hbm, eu_hbm, o_u),
                                (g_idx_hbm, eg_hbm, o_g),
                                (ti_idx_hbm, ei_hbm, o_ti),
                                (tc_idx_hbm, ec_hbm, o_tc)):
      pltpu.sync_copy(idx_hbm.at[pl.ds(bs, bpw_s)], idx_s)
      pltpu.async_copy(table.at[idx_s], rows_s, sem_s).wait()
      pltpu.sync_copy(rows_s, out.at[pl.ds(bs, bpw_s)])

  return gather_kernel(E_item, E_cate, E_user, E_gender,
                       hist_item_idx, hist_cate_idx, user_idx, gender_idx,
                       target_item_idx, target_cate_idx)


def _attn_body(hi_ref, hc_ref, u_ref, g_ref, ti_ref, tc_ref, len_ref,
               Wa1_ref, ba1_ref, Wa2_ref, ba2_ref, Wa3_ref, ba3_ref,
               join_ref, *, BB, T, D2):
  f32 = jnp.float32
  D = D2 // 2
  Wa1 = Wa1_ref[...]
  Whd = Wa1[D2:2 * D2, :] - Wa1[2 * D2:3 * D2, :]
  Wm = Wa1[3 * D2:4 * D2, :]
  W1c = jnp.concatenate([Whd, Wm], axis=0)
  Wqd = Wa1[0:D2, :] + Wa1[2 * D2:3 * D2, :]

  q = jnp.concatenate([ti_ref[...][:, :D], tc_ref[...][:, :D]],
                      axis=-1).astype(f32)
  utgc = jnp.concatenate([u_ref[...][:, :D].astype(f32),
                          g_ref[...][:, :D].astype(f32), q], axis=-1)
  h3 = jnp.concatenate([hi_ref[...][:, :D], hc_ref[...][:, :D]],
                       axis=-1).astype(f32).reshape(BB, T, D2)
  X = jnp.concatenate([h3, h3 * q[:, None, :]], axis=-1).reshape(BB * T, 2 * D2)
  Z1 = jnp.dot(X, W1c, preferred_element_type=f32)
  qa = jnp.dot(q, Wqd, preferred_element_type=f32) + ba1_ref[...]
  A1 = jax.nn.sigmoid(Z1.reshape(BB, T, 80) + qa[:, None, :]).reshape(BB * T, 80)
  A2 = jax.nn.sigmoid(
      jnp.dot(A1, Wa2_ref[...], preferred_element_type=f32) + ba2_ref[...])
  s = jnp.dot(A2, Wa3_ref[...], preferred_element_type=f32) + ba3_ref[0, 0]
  s = s.reshape(BB, T) * (1.0 / jnp.sqrt(jnp.float32(D2)))
  pos = jax.lax.broadcasted_iota(jnp.int32, (BB, T), 1)
  s = jnp.where(pos < len_ref[...], s, jnp.float32(-(2.0 ** 32) + 1.0))
  s = s - jnp.max(s, axis=-1, keepdims=True)
  e = jnp.exp(s)
  w = e / jnp.sum(e, axis=-1, keepdims=True)
  rows = [jnp.dot(w[b:b + 1, :], h3[b], preferred_element_type=f32)
          for b in range(BB)]
  attn = jnp.concatenate(rows, axis=0)
  join_ref[...] = jnp.concatenate([utgc, attn], axis=-1)


def _fc_body(join_ref, gamma_ref, beta_ref, Wf1_ref, bf1_ref, a1_ref,
             Wf2_ref, bf2_ref, a2_ref, Wf3_ref, bf3_ref,
             out_ref, logit_ref):
  f32 = jnp.float32
  x = join_ref[...]
  mean = jnp.mean(x, axis=0, keepdims=True)
  var = jnp.mean((x - mean) ** 2, axis=0, keepdims=True)
  xn = (x - mean) / jnp.sqrt(var + 1e-3) * gamma_ref[...] + beta_ref[...]
  h = jnp.dot(xn, Wf1_ref[...], preferred_element_type=f32) + bf1_ref[...]
  h = jnp.maximum(h, 0.0)
  h = h + a1_ref[...] * jnp.minimum(h, 0.0)
  h2 = jnp.dot(h, Wf2_ref[...], preferred_element_type=f32) + bf2_ref[...]
  h2 = jnp.maximum(h2, 0.0)
  h2 = h2 + a2_ref[...] * jnp.minimum(h2, 0.0)
  logit = jnp.dot(h2, Wf3_ref[...], preferred_element_type=f32) + bf3_ref[...]
  m = jnp.max(logit, axis=-1, keepdims=True)
  e = jnp.exp(logit - m)
  out_ref[...] = e / jnp.sum(e, axis=-1, keepdims=True)
  logit_ref[...] = logit


def kernel(E_user, E_gender, E_item, E_cate, Wa1, ba1, Wa2, ba2, Wa3, ba3,
           gamma, beta, Wf1, bf1, alpha1, Wf2, bf2, alpha2, Wf3, bf3,
           user_id, gender, target_item_id, target_cate_id,
           hist_item_id, hist_cate_id, length):
  B, T = hist_item_id.shape
  D = E_item.shape[1]
  D2 = 2 * D
  f32 = jnp.float32
  i32 = jnp.int32

  pad = lambda E: jnp.pad(E, ((0, 0), (0, D)))
  Ei_p, Ec_p, Eu_p, Eg_p = pad(E_item), pad(E_cate), pad(E_user), pad(E_gender)
  hi_idx = hist_item_id.reshape(B * T).astype(i32)
  hc_idx = hist_cate_id.reshape(B * T).astype(i32)
  u_idx = user_id.astype(i32)
  g_idx = gender.astype(i32)
  ti_idx = target_item_id.astype(i32)
  tc_idx = target_cate_id.astype(i32)

  BB = 16
  len_i = length.astype(i32).reshape(B, 1)

  NCH = 8
  Bc = B // NCH
  BTc = Bc * T
  attn_call = pl.pallas_call(
      functools.partial(_attn_body, BB=BB, T=T, D2=D2),
      grid=(Bc // BB,),
      in_specs=[
          pl.BlockSpec((BB * T, D2), lambda i: (i, 0)),
          pl.BlockSpec((BB * T, D2), lambda i: (i, 0)),
          pl.BlockSpec((BB, D2), lambda i: (i, 0)),
          pl.BlockSpec((BB, D2), lambda i: (i, 0)),
          pl.BlockSpec((BB, D2), lambda i: (i, 0)),
          pl.BlockSpec((BB, D2), lambda i: (i, 0)),
          pl.BlockSpec((BB, 1), lambda i: (i, 0)),
          pl.BlockSpec((4 * D2, 80), lambda i: (0, 0)),
          pl.BlockSpec((1, 80), lambda i: (0, 0)),
          pl.BlockSpec((80, 40), lambda i: (0, 0)),
          pl.BlockSpec((1, 40), lambda i: (0, 0)),
          pl.BlockSpec((40, 1), lambda i: (0, 0)),
          pl.BlockSpec((1, 1), lambda i: (0, 0)),
      ],
      out_specs=pl.BlockSpec((BB, 6 * D), lambda i: (i, 0)),
      out_shape=jax.ShapeDtypeStruct((Bc, 6 * D), f32),
  )
  joins = []
  for k in range(NCH):
    st, sb = k * BTc, k * Bc
    hi, hc, u, g, ti, tc = _sc_gather(
        Ei_p, Ec_p, Eu_p, Eg_p,
        hi_idx[st:st + BTc], hc_idx[st:st + BTc],
        u_idx[sb:sb + Bc], g_idx[sb:sb + Bc],
        ti_idx[sb:sb + Bc], tc_idx[sb:sb + Bc])
    joins.append(attn_call(
        hi, hc, u, g, ti, tc, len_i[sb:sb + Bc], Wa1, ba1.reshape(1, -1),
        Wa2, ba2.reshape(1, -1), Wa3, ba3.reshape(1, 1)))
  join = jnp.concatenate(joins, axis=0)

  out, logit = pl.pallas_call(
      _fc_body,
      out_shape=(jax.ShapeDtypeStruct((B, 2), f32),
                 jax.ShapeDtypeStruct((B, 2), f32)),
  )(join, gamma.reshape(1, -1), beta.reshape(1, -1),
    Wf1, bf1.reshape(1, -1), alpha1.reshape(1, -1),
    Wf2, bf2.reshape(1, -1), alpha2.reshape(1, -1),
    Wf3, bf3.reshape(1, -1))
  return out, logit


# speedup vs baseline: 7.8146x; 1.0344x over previous
import functools

import jax
import jax.numpy as jnp
from jax.experimental import layout as jlayout
from jax.experimental import pallas as pl
from jax.experimental.pallas import tpu as pltpu
from jax.experimental.pallas import tpu_sc as plsc


def _sublane_tiled(x):
  lay = jlayout.Layout(major_to_minor=(0, 1), tiling=((16,),))
  return jlayout.with_layout_constraint(x, lay)


def _sc_gather(E_item, E_cate, E_user, E_gender,
               hist_item_idx, hist_cate_idx, user_idx, gender_idx,
               target_item_idx, target_cate_idx):
  BT = hist_item_idx.shape[0]
  Bn = user_idx.shape[0]
  DP = E_item.shape[1]
  dt = E_item.dtype
  i32 = jnp.int32
  NC, NS = 2, 16
  NW = NC * NS
  C = 200
  bpw_h = BT // NW
  n_h = bpw_h // C
  bpw_s = Bn // NW
  mesh = plsc.VectorSubcoreMesh(core_axis_name="c", subcore_axis_name="s")

  @functools.partial(
      pl.kernel,
      out_type=(
          jax.ShapeDtypeStruct((BT, DP), dt),
          jax.ShapeDtypeStruct((BT, DP), dt),
          jax.ShapeDtypeStruct((Bn, DP), dt),
          jax.ShapeDtypeStruct((Bn, DP), dt),
          jax.ShapeDtypeStruct((Bn, DP), dt),
          jax.ShapeDtypeStruct((Bn, DP), dt),
      ),
      mesh=mesh,
      scratch_types=[
          pltpu.VMEM((C,), i32),
          pltpu.VMEM((C,), i32),
          pltpu.VMEM((C,), i32),
          pltpu.VMEM((C,), i32),
          pltpu.VMEM((C, DP), dt),
          pltpu.VMEM((C, DP), dt),
          pltpu.VMEM((C, DP), dt),
          pltpu.VMEM((C, DP), dt),
          pltpu.VMEM((bpw_s,), i32),
          pltpu.VMEM((bpw_s, DP), dt),
          pltpu.SemaphoreType.DMA,
          pltpu.SemaphoreType.DMA,
          pltpu.SemaphoreType.DMA,
      ],
  )
  def gather_kernel(ei_hbm, ec_hbm, eu_hbm, eg_hbm,
                    hi_idx_hbm, hc_idx_hbm, u_idx_hbm, g_idx_hbm,
                    ti_idx_hbm, tc_idx_hbm,
                    o_hi, o_hc, o_u, o_g, o_ti, o_tc,
                    ii0, ic0, ii1, ic1, ri0, rc0, ri1, rc1, idx_s, rows_s,
                    sem0, sem1, sem_s):
    wid = jax.lax.axis_index("s") * NC + jax.lax.axis_index("c")
    base_h = wid * bpw_h
    bufs = ((ii0, ic0, ri0, rc0, sem0),
            (ii1, ic1, ri1, rc1, sem1))

    def load_start(ci, s):
      ii, ic, ri, rc, sem = bufs[s]
      b = base_h + ci * C
      pltpu.sync_copy(hi_idx_hbm.at[pl.ds(b, C)], ii)
      pltpu.sync_copy(hc_idx_hbm.at[pl.ds(b, C)], ic)
      pltpu.async_copy(ei_hbm.at[ii], ri, sem)
      pltpu.async_copy(ec_hbm.at[ic], rc, sem)

    def drain_store(ci, s):
      ii, ic, ri, rc, sem = bufs[s]
      b = base_h + ci * C
      pltpu.make_async_copy(ei_hbm.at[ii], ri, sem).wait()
      pltpu.make_async_copy(ec_hbm.at[ic], rc, sem).wait()
      pltpu.sync_copy(ri, o_hi.at[pl.ds(b, C)])
      pltpu.sync_copy(rc, o_hc.at[pl.ds(b, C)])

    load_start(0, 0)
    @pl.loop(0, n_h // 2 - 1)
    def _(j):
      c = 2 * j
      load_start(c + 1, 1)
      drain_store(c, 0)
      load_start(c + 2, 0)
      drain_store(c + 1, 1)
    load_start(n_h - 1, 1)
    drain_store(n_h - 2, 0)
    drain_store(n_h - 1, 1)

    bs = wid * bpw_s
    for idx_hbm, table, out in ((u_idx_hbm, eu_hbm, o_u),
                                (g_idx_hbm, eg_hbm, o_g),
                                (ti_idx_hbm, ei_hbm, o_ti),
                                (tc_idx_hbm, ec_hbm, o_tc)):
      pltpu.sync_copy(idx_hbm.at[pl.ds(bs, bpw_s)], idx_s)
      pltpu.async_copy(table.at[idx_s], rows_s, sem_s).wait()
      pltpu.sync_copy(rows_s, out.at[pl.ds(bs, bpw_s)])

  return gather_kernel(E_item, E_cate, E_user, E_gender,
                       hist_item_idx, hist_cate_idx, user_idx, gender_idx,
                       target_item_idx, target_cate_idx)


def _attn_body(hi_ref, hc_ref, u_ref, g_ref, ti_ref, tc_ref, len_ref,
               Wa1_ref, ba1_ref, Wa2_ref, ba2_ref, Wa3_ref, ba3_ref,
               join_ref, *, BB, T, D2):
  f32 = jnp.float32
  D = D2 // 2
  Wa1 = Wa1_ref[...]
  Whd = Wa1[D2:2 * D2, :] - Wa1[2 * D2:3 * D2, :]
  Wm = Wa1[3 * D2:4 * D2, :]
  W1c = jnp.concatenate([Whd, Wm], axis=0)
  Wqd = Wa1[0:D2, :] + Wa1[2 * D2:3 * D2, :]

  q = jnp.concatenate([ti_ref[...], tc_ref[...]], axis=-1)
  utgc = jnp.concatenate([u_ref[...], g_ref[...], q], axis=-1)
  h3 = jnp.concatenate([hi_ref[...], hc_ref[...]],
                       axis=-1).reshape(BB, T, D2)
  X = jnp.concatenate([h3, h3 * q[:, None, :]], axis=-1).reshape(BB * T, 2 * D2)
  Z1 = jnp.dot(X, W1c, preferred_element_type=f32)
  qa = jnp.dot(q, Wqd, preferred_element_type=f32) + ba1_ref[...]
  A1 = jax.nn.sigmoid(Z1.reshape(BB, T, 80) + qa[:, None, :]).reshape(BB * T, 80)
  A2 = jax.nn.sigmoid(
      jnp.dot(A1, Wa2_ref[...], preferred_element_type=f32) + ba2_ref[...])
  s = jnp.dot(A2, Wa3_ref[...], preferred_element_type=f32) + ba3_ref[0, 0]
  s = s.reshape(BB, T) * (1.0 / jnp.sqrt(jnp.float32(D2)))
  pos = jax.lax.broadcasted_iota(jnp.int32, (BB, T), 1)
  s = jnp.where(pos < len_ref[...], s, jnp.float32(-(2.0 ** 32) + 1.0))
  s = s - jnp.max(s, axis=-1, keepdims=True)
  e = jnp.exp(s)
  w = e / jnp.sum(e, axis=-1, keepdims=True)
  rows = [jnp.dot(w[b:b + 1, :], h3[b], preferred_element_type=f32)
          for b in range(BB)]
  attn = jnp.concatenate(rows, axis=0)
  join_ref[...] = jnp.concatenate([utgc, attn], axis=-1)


def _fc_body(join_ref, gamma_ref, beta_ref, Wf1_ref, bf1_ref, a1_ref,
             Wf2_ref, bf2_ref, a2_ref, Wf3_ref, bf3_ref,
             out_ref, logit_ref):
  f32 = jnp.float32
  x = join_ref[...]
  mean = jnp.mean(x, axis=0, keepdims=True)
  var = jnp.mean((x - mean) ** 2, axis=0, keepdims=True)
  xn = (x - mean) / jnp.sqrt(var + 1e-3) * gamma_ref[...] + beta_ref[...]
  h = jnp.dot(xn, Wf1_ref[...], preferred_element_type=f32) + bf1_ref[...]
  h = jnp.maximum(h, 0.0)
  h = h + a1_ref[...] * jnp.minimum(h, 0.0)
  h2 = jnp.dot(h, Wf2_ref[...], preferred_element_type=f32) + bf2_ref[...]
  h2 = jnp.maximum(h2, 0.0)
  h2 = h2 + a2_ref[...] * jnp.minimum(h2, 0.0)
  logit = jnp.dot(h2, Wf3_ref[...], preferred_element_type=f32) + bf3_ref[...]
  m = jnp.max(logit, axis=-1, keepdims=True)
  e = jnp.exp(logit - m)
  out_ref[...] = e / jnp.sum(e, axis=-1, keepdims=True)
  logit_ref[...] = logit


def kernel(E_user, E_gender, E_item, E_cate, Wa1, ba1, Wa2, ba2, Wa3, ba3,
           gamma, beta, Wf1, bf1, alpha1, Wf2, bf2, alpha2, Wf3, bf3,
           user_id, gender, target_item_id, target_cate_id,
           hist_item_id, hist_cate_id, length):
  B, T = hist_item_id.shape
  D = E_item.shape[1]
  D2 = 2 * D
  f32 = jnp.float32
  i32 = jnp.int32

  Ei_p, Ec_p, Eu_p, Eg_p = (_sublane_tiled(E_item), _sublane_tiled(E_cate),
                            _sublane_tiled(E_user), _sublane_tiled(E_gender))
  hi_idx = hist_item_id.reshape(B * T).astype(i32)
  hc_idx = hist_cate_id.reshape(B * T).astype(i32)
  u_idx = user_id.astype(i32)
  g_idx = gender.astype(i32)
  ti_idx = target_item_id.astype(i32)
  tc_idx = target_cate_id.astype(i32)

  BB = 16
  len_i = length.astype(i32).reshape(B, 1)

  NCH = 8
  Bc = B // NCH
  BTc = Bc * T
  attn_call = pl.pallas_call(
      functools.partial(_attn_body, BB=BB, T=T, D2=D2),
      grid=(Bc // BB,),
      in_specs=[
          pl.BlockSpec((BB * T, D), lambda i: (i, 0)),
          pl.BlockSpec((BB * T, D), lambda i: (i, 0)),
          pl.BlockSpec((BB, D), lambda i: (i, 0)),
          pl.BlockSpec((BB, D), lambda i: (i, 0)),
          pl.BlockSpec((BB, D), lambda i: (i, 0)),
          pl.BlockSpec((BB, D), lambda i: (i, 0)),
          pl.BlockSpec((BB, 1), lambda i: (i, 0)),
          pl.BlockSpec((4 * D2, 80), lambda i: (0, 0)),
          pl.BlockSpec((1, 80), lambda i: (0, 0)),
          pl.BlockSpec((80, 40), lambda i: (0, 0)),
          pl.BlockSpec((1, 40), lambda i: (0, 0)),
          pl.BlockSpec((40, 1), lambda i: (0, 0)),
          pl.BlockSpec((1, 1), lambda i: (0, 0)),
      ],
      out_specs=pl.BlockSpec((BB, 6 * D), lambda i: (i, 0)),
      out_shape=jax.ShapeDtypeStruct((Bc, 6 * D), f32),
  )
  joins = []
  for k in range(NCH):
    st, sb = k * BTc, k * Bc
    hi, hc, u, g, ti, tc = _sc_gather(
        Ei_p, Ec_p, Eu_p, Eg_p,
        hi_idx[st:st + BTc], hc_idx[st:st + BTc],
        u_idx[sb:sb + Bc], g_idx[sb:sb + Bc],
        ti_idx[sb:sb + Bc], tc_idx[sb:sb + Bc])
    joins.append(attn_call(
        hi, hc, u, g, ti, tc, len_i[sb:sb + Bc], Wa1, ba1.reshape(1, -1),
        Wa2, ba2.reshape(1, -1), Wa3, ba3.reshape(1, 1)))
  join = jnp.concatenate(joins, axis=0)

  out, logit = pl.pallas_call(
      _fc_body,
      out_shape=(jax.ShapeDtypeStruct((B, 2), f32),
                 jax.ShapeDtypeStruct((B, 2), f32)),
  )(join, gamma.reshape(1, -1), beta.reshape(1, -1),
    Wf1, bf1.reshape(1, -1), alpha1.reshape(1, -1),
    Wf2, bf2.reshape(1, -1), alpha2.reshape(1, -1),
    Wf3, bf3.reshape(1, -1))
  return out, logit


# speedup vs baseline: 8.3695x; 1.0710x over previous
import functools

import jax
import jax.numpy as jnp
from jax.experimental import layout as jlayout
from jax.experimental import pallas as pl
from jax.experimental.pallas import tpu as pltpu
from jax.experimental.pallas import tpu_sc as plsc


def _sublane_tiled(x):
  lay = jlayout.Layout(major_to_minor=(0, 1), tiling=((16,),))
  return jlayout.with_layout_constraint(x, lay)


def _sc_gather(E_item, E_cate, E_user, E_gender,
               hist_item_idx, hist_cate_idx, user_idx, gender_idx,
               target_item_idx, target_cate_idx):
  BT = hist_item_idx.shape[0]
  Bn = user_idx.shape[0]
  DP = E_item.shape[1]
  dt = E_item.dtype
  i32 = jnp.int32
  NC, NS = 2, 16
  NW = NC * NS
  C = 200
  bpw_h = BT // NW
  n_h = bpw_h // C
  bpw_s = Bn // NW
  mesh = plsc.VectorSubcoreMesh(core_axis_name="c", subcore_axis_name="s")

  @functools.partial(
      pl.kernel,
      out_type=(
          jax.ShapeDtypeStruct((BT, DP), dt),
          jax.ShapeDtypeStruct((BT, DP), dt),
          jax.ShapeDtypeStruct((Bn, DP), dt),
          jax.ShapeDtypeStruct((Bn, DP), dt),
          jax.ShapeDtypeStruct((Bn, DP), dt),
          jax.ShapeDtypeStruct((Bn, DP), dt),
      ),
      mesh=mesh,
      scratch_types=[
          pltpu.VMEM((C,), i32),
          pltpu.VMEM((C,), i32),
          pltpu.VMEM((C,), i32),
          pltpu.VMEM((C,), i32),
          pltpu.VMEM((C, DP), dt),
          pltpu.VMEM((C, DP), dt),
          pltpu.VMEM((C, DP), dt),
          pltpu.VMEM((C, DP), dt),
          pltpu.VMEM((bpw_s,), i32),
          pltpu.VMEM((bpw_s, DP), dt),
          pltpu.SemaphoreType.DMA,
          pltpu.SemaphoreType.DMA,
          pltpu.SemaphoreType.DMA,
      ],
  )
  def gather_kernel(ei_hbm, ec_hbm, eu_hbm, eg_hbm,
                    hi_idx_hbm, hc_idx_hbm, u_idx_hbm, g_idx_hbm,
                    ti_idx_hbm, tc_idx_hbm,
                    o_hi, o_hc, o_u, o_g, o_ti, o_tc,
                    ii0, ic0, ii1, ic1, ri0, rc0, ri1, rc1, idx_s, rows_s,
                    sem0, sem1, sem_s):
    wid = jax.lax.axis_index("s") * NC + jax.lax.axis_index("c")
    base_h = wid * bpw_h
    bufs = ((ii0, ic0, ri0, rc0, sem0),
            (ii1, ic1, ri1, rc1, sem1))

    def load_start(ci, s):
      ii, ic, ri, rc, sem = bufs[s]
      b = base_h + ci * C
      pltpu.sync_copy(hi_idx_hbm.at[pl.ds(b, C)], ii)
      pltpu.sync_copy(hc_idx_hbm.at[pl.ds(b, C)], ic)
      pltpu.async_copy(ei_hbm.at[ii], ri, sem)
      pltpu.async_copy(ec_hbm.at[ic], rc, sem)

    def drain_store(ci, s):
      ii, ic, ri, rc, sem = bufs[s]
      b = base_h + ci * C
      pltpu.make_async_copy(ei_hbm.at[ii], ri, sem).wait()
      pltpu.make_async_copy(ec_hbm.at[ic], rc, sem).wait()
      pltpu.sync_copy(ri, o_hi.at[pl.ds(b, C)])
      pltpu.sync_copy(rc, o_hc.at[pl.ds(b, C)])

    load_start(0, 0)
    @pl.loop(0, n_h // 2 - 1)
    def _(j):
      c = 2 * j
      load_start(c + 1, 1)
      drain_store(c, 0)
      load_start(c + 2, 0)
      drain_store(c + 1, 1)
    load_start(n_h - 1, 1)
    drain_store(n_h - 2, 0)
    drain_store(n_h - 1, 1)

    bs = wid * bpw_s
    for idx_hbm, table, out in ((u_idx_hbm, eu_hbm, o_u),
                                (g_idx_hbm, eg_hbm, o_g),
                                (ti_idx_hbm, ei_hbm, o_ti),
                                (tc_idx_hbm, ec_hbm, o_tc)):
      pltpu.sync_copy(idx_hbm.at[pl.ds(bs, bpw_s)], idx_s)
      pltpu.async_copy(table.at[idx_s], rows_s, sem_s).wait()
      pltpu.sync_copy(rows_s, out.at[pl.ds(bs, bpw_s)])

  return gather_kernel(E_item, E_cate, E_user, E_gender,
                       hist_item_idx, hist_cate_idx, user_idx, gender_idx,
                       target_item_idx, target_cate_idx)


def _attn_body(hi_ref, hc_ref, u_ref, g_ref, ti_ref, tc_ref, len_ref,
               Wa1_ref, ba1_ref, Wa2_ref, ba2_ref, Wa3_ref, ba3_ref,
               join_ref, *, BB, T, D2):
  f32 = jnp.float32
  D = D2 // 2
  Wa1 = Wa1_ref[...]
  Whd = Wa1[D2:2 * D2, :] - Wa1[2 * D2:3 * D2, :]
  Wm = Wa1[3 * D2:4 * D2, :]
  W1c = jnp.concatenate([Whd, Wm], axis=0)
  Wqd = Wa1[0:D2, :] + Wa1[2 * D2:3 * D2, :]

  bf16 = jnp.bfloat16
  q = jnp.concatenate([ti_ref[...], tc_ref[...]], axis=-1)
  utgc = jnp.concatenate([u_ref[...], g_ref[...], q], axis=-1)
  h3 = jnp.concatenate([hi_ref[...], hc_ref[...]],
                       axis=-1).astype(bf16).reshape(BB, T, D2)
  X = jnp.concatenate([h3, h3 * q.astype(bf16)[:, None, :]],
                      axis=-1).reshape(BB * T, 2 * D2)
  Z1 = jnp.dot(X, W1c.astype(bf16), preferred_element_type=f32)
  qa = jnp.dot(q, Wqd, preferred_element_type=f32) + ba1_ref[...]
  A1 = jax.nn.sigmoid(Z1.reshape(BB, T, 80) + qa[:, None, :])
  A1 = A1.reshape(BB * T, 80).astype(bf16)
  A2 = jax.nn.sigmoid(
      jnp.dot(A1, Wa2_ref[...].astype(bf16),
              preferred_element_type=f32) + ba2_ref[...]).astype(bf16)
  s = jnp.dot(A2, Wa3_ref[...].astype(bf16),
              preferred_element_type=f32) + ba3_ref[0, 0]
  s = s.reshape(BB, T) * (1.0 / jnp.sqrt(jnp.float32(D2)))
  pos = jax.lax.broadcasted_iota(jnp.int32, (BB, T), 1)
  s = jnp.where(pos < len_ref[...], s, jnp.float32(-(2.0 ** 32) + 1.0))
  s = s - jnp.max(s, axis=-1, keepdims=True)
  e = jnp.exp(s)
  w = e / jnp.sum(e, axis=-1, keepdims=True)
  rows = [jnp.dot(w[b:b + 1, :].astype(bf16), h3[b],
                  preferred_element_type=f32) for b in range(BB)]
  attn = jnp.concatenate(rows, axis=0)
  join_ref[...] = jnp.concatenate([utgc, attn], axis=-1)


def _fc_body(join_ref, gamma_ref, beta_ref, Wf1_ref, bf1_ref, a1_ref,
             Wf2_ref, bf2_ref, a2_ref, Wf3_ref, bf3_ref,
             out_ref, logit_ref):
  f32 = jnp.float32
  x = join_ref[...]
  mean = jnp.mean(x, axis=0, keepdims=True)
  var = jnp.mean((x - mean) ** 2, axis=0, keepdims=True)
  xn = (x - mean) / jnp.sqrt(var + 1e-3) * gamma_ref[...] + beta_ref[...]
  h = jnp.dot(xn, Wf1_ref[...], preferred_element_type=f32) + bf1_ref[...]
  h = jnp.maximum(h, 0.0)
  h = h + a1_ref[...] * jnp.minimum(h, 0.0)
  h2 = jnp.dot(h, Wf2_ref[...], preferred_element_type=f32) + bf2_ref[...]
  h2 = jnp.maximum(h2, 0.0)
  h2 = h2 + a2_ref[...] * jnp.minimum(h2, 0.0)
  logit = jnp.dot(h2, Wf3_ref[...], preferred_element_type=f32) + bf3_ref[...]
  m = jnp.max(logit, axis=-1, keepdims=True)
  e = jnp.exp(logit - m)
  out_ref[...] = e / jnp.sum(e, axis=-1, keepdims=True)
  logit_ref[...] = logit


def kernel(E_user, E_gender, E_item, E_cate, Wa1, ba1, Wa2, ba2, Wa3, ba3,
           gamma, beta, Wf1, bf1, alpha1, Wf2, bf2, alpha2, Wf3, bf3,
           user_id, gender, target_item_id, target_cate_id,
           hist_item_id, hist_cate_id, length):
  B, T = hist_item_id.shape
  D = E_item.shape[1]
  D2 = 2 * D
  f32 = jnp.float32
  i32 = jnp.int32

  Ei_p, Ec_p, Eu_p, Eg_p = (_sublane_tiled(E_item), _sublane_tiled(E_cate),
                            _sublane_tiled(E_user), _sublane_tiled(E_gender))
  hi_idx = hist_item_id.reshape(B * T).astype(i32)
  hc_idx = hist_cate_id.reshape(B * T).astype(i32)
  u_idx = user_id.astype(i32)
  g_idx = gender.astype(i32)
  ti_idx = target_item_id.astype(i32)
  tc_idx = target_cate_id.astype(i32)

  BB = 64
  len_i = length.astype(i32).reshape(B, 1)

  NCH = 8
  Bc = B // NCH
  BTc = Bc * T
  attn_call = pl.pallas_call(
      functools.partial(_attn_body, BB=BB, T=T, D2=D2),
      grid=(Bc // BB,),
      in_specs=[
          pl.BlockSpec((BB * T, D), lambda i: (i, 0)),
          pl.BlockSpec((BB * T, D), lambda i: (i, 0)),
          pl.BlockSpec((BB, D), lambda i: (i, 0)),
          pl.BlockSpec((BB, D), lambda i: (i, 0)),
          pl.BlockSpec((BB, D), lambda i: (i, 0)),
          pl.BlockSpec((BB, D), lambda i: (i, 0)),
          pl.BlockSpec((BB, 1), lambda i: (i, 0)),
          pl.BlockSpec((4 * D2, 80), lambda i: (0, 0)),
          pl.BlockSpec((1, 80), lambda i: (0, 0)),
          pl.BlockSpec((80, 40), lambda i: (0, 0)),
          pl.BlockSpec((1, 40), lambda i: (0, 0)),
          pl.BlockSpec((40, 1), lambda i: (0, 0)),
          pl.BlockSpec((1, 1), lambda i: (0, 0)),
      ],
      out_specs=pl.BlockSpec((BB, 6 * D), lambda i: (i, 0)),
      out_shape=jax.ShapeDtypeStruct((Bc, 6 * D), f32),
  )
  joins = []
  for k in range(NCH):
    st, sb = k * BTc, k * Bc
    hi, hc, u, g, ti, tc = _sc_gather(
        Ei_p, Ec_p, Eu_p, Eg_p,
        hi_idx[st:st + BTc], hc_idx[st:st + BTc],
        u_idx[sb:sb + Bc], g_idx[sb:sb + Bc],
        ti_idx[sb:sb + Bc], tc_idx[sb:sb + Bc])
    joins.append(attn_call(
        hi, hc, u, g, ti, tc, len_i[sb:sb + Bc], Wa1, ba1.reshape(1, -1),
        Wa2, ba2.reshape(1, -1), Wa3, ba3.reshape(1, 1)))
  join = jnp.concatenate(joins, axis=0)

  out, logit = pl.pallas_call(
      _fc_body,
      out_shape=(jax.ShapeDtypeStruct((B, 2), f32),
                 jax.ShapeDtypeStruct((B, 2), f32)),
  )(join, gamma.reshape(1, -1), beta.reshape(1, -1),
    Wf1, bf1.reshape(1, -1), alpha1.reshape(1, -1),
    Wf2, bf2.reshape(1, -1), alpha2.reshape(1, -1),
    Wf3, bf3.reshape(1, -1))
  return out, logit


# speedup vs baseline: 8.7269x; 1.0427x over previous
import functools

import jax
import jax.numpy as jnp
from jax.experimental import layout as jlayout
from jax.experimental import pallas as pl
from jax.experimental.pallas import tpu as pltpu
from jax.experimental.pallas import tpu_sc as plsc


def _sublane_tiled(x):
  lay = jlayout.Layout(major_to_minor=(0, 1), tiling=((16,),))
  return jlayout.with_layout_constraint(x, lay)


def _sc_gather(E_item, E_cate, E_user, E_gender,
               hist_item_idx, hist_cate_idx, user_idx, gender_idx,
               target_item_idx, target_cate_idx):
  BT = hist_item_idx.shape[0]
  Bn = user_idx.shape[0]
  DP = E_item.shape[1]
  dt = E_item.dtype
  i32 = jnp.int32
  NC, NS = 2, 16
  NW = NC * NS
  C = 200
  bpw_h = BT // NW
  n_h = bpw_h // C
  bpw_s = Bn // NW
  mesh = plsc.VectorSubcoreMesh(core_axis_name="c", subcore_axis_name="s")

  @functools.partial(
      pl.kernel,
      out_type=(
          jax.ShapeDtypeStruct((BT, DP), dt),
          jax.ShapeDtypeStruct((BT, DP), dt),
          jax.ShapeDtypeStruct((Bn, DP), dt),
          jax.ShapeDtypeStruct((Bn, DP), dt),
          jax.ShapeDtypeStruct((Bn, DP), dt),
          jax.ShapeDtypeStruct((Bn, DP), dt),
      ),
      mesh=mesh,
      scratch_types=[
          pltpu.VMEM((C,), i32),
          pltpu.VMEM((C,), i32),
          pltpu.VMEM((C,), i32),
          pltpu.VMEM((C,), i32),
          pltpu.VMEM((C, DP), dt),
          pltpu.VMEM((C, DP), dt),
          pltpu.VMEM((C, DP), dt),
          pltpu.VMEM((C, DP), dt),
          pltpu.VMEM((bpw_s,), i32),
          pltpu.VMEM((bpw_s, DP), dt),
          pltpu.SemaphoreType.DMA,
          pltpu.SemaphoreType.DMA,
          pltpu.SemaphoreType.DMA,
      ],
  )
  def gather_kernel(ei_hbm, ec_hbm, eu_hbm, eg_hbm,
                    hi_idx_hbm, hc_idx_hbm, u_idx_hbm, g_idx_hbm,
                    ti_idx_hbm, tc_idx_hbm,
                    o_hi, o_hc, o_u, o_g, o_ti, o_tc,
                    ii0, ic0, ii1, ic1, ri0, rc0, ri1, rc1, idx_s, rows_s,
                    sem0, sem1, sem_s):
    wid = jax.lax.axis_index("s") * NC + jax.lax.axis_index("c")
    base_h = wid * bpw_h
    bufs = ((ii0, ic0, ri0, rc0, sem0),
            (ii1, ic1, ri1, rc1, sem1))

    def load_start(ci, s):
      ii, ic, ri, rc, sem = bufs[s]
      b = base_h + ci * C
      pltpu.sync_copy(hi_idx_hbm.at[pl.ds(b, C)], ii)
      pltpu.sync_copy(hc_idx_hbm.at[pl.ds(b, C)], ic)
      pltpu.async_copy(ei_hbm.at[ii], ri, sem)
      pltpu.async_copy(ec_hbm.at[ic], rc, sem)

    def drain_store(ci, s):
      ii, ic, ri, rc, sem = bufs[s]
      b = base_h + ci * C
      pltpu.make_async_copy(ei_hbm.at[ii], ri, sem).wait()
      pltpu.make_async_copy(ec_hbm.at[ic], rc, sem).wait()
      pltpu.sync_copy(ri, o_hi.at[pl.ds(b, C)])
      pltpu.sync_copy(rc, o_hc.at[pl.ds(b, C)])

    load_start(0, 0)
    @pl.loop(0, n_h // 2 - 1)
    def _(j):
      c = 2 * j
      load_start(c + 1, 1)
      drain_store(c, 0)
      load_start(c + 2, 0)
      drain_store(c + 1, 1)
    load_start(n_h - 1, 1)
    drain_store(n_h - 2, 0)
    drain_store(n_h - 1, 1)

    bs = wid * bpw_s
    for idx_hbm, table, out in ((u_idx_hbm, eu_hbm, o_u),
                                (g_idx_hbm, eg_hbm, o_g),
                                (ti_idx_hbm, ei_hbm, o_ti),
                                (tc_idx_hbm, ec_hbm, o_tc)):
      pltpu.sync_copy(idx_hbm.at[pl.ds(bs, bpw_s)], idx_s)
      pltpu.async_copy(table.at[idx_s], rows_s, sem_s).wait()
      pltpu.sync_copy(rows_s, out.at[pl.ds(bs, bpw_s)])

  return gather_kernel(E_item, E_cate, E_user, E_gender,
                       hist_item_idx, hist_cate_idx, user_idx, gender_idx,
                       target_item_idx, target_cate_idx)


def _attn_body(hi_ref, hc_ref, u_ref, g_ref, ti_ref, tc_ref, len_ref,
               Wa1_ref, ba1_ref, Wa2_ref, ba2_ref, Wa3_ref, ba3_ref,
               join_ref, *, BB, T, D2):
  f32 = jnp.float32
  D = D2 // 2
  Wa1 = Wa1_ref[...]
  Whd = Wa1[D2:2 * D2, :] - Wa1[2 * D2:3 * D2, :]
  Wm = Wa1[3 * D2:4 * D2, :]
  W1c = jnp.concatenate([Whd, Wm], axis=0)
  Wqd = Wa1[0:D2, :] + Wa1[2 * D2:3 * D2, :]

  bf16 = jnp.bfloat16
  q = jnp.concatenate([ti_ref[...], tc_ref[...]], axis=-1)
  utgc = jnp.concatenate([u_ref[...], g_ref[...], q], axis=-1)
  h3 = jnp.concatenate([hi_ref[...], hc_ref[...]],
                       axis=-1).astype(bf16).reshape(BB, T, D2)
  X = jnp.concatenate([h3, h3 * q.astype(bf16)[:, None, :]],
                      axis=-1).reshape(BB * T, 2 * D2)
  Z1 = jnp.dot(X, W1c.astype(bf16), preferred_element_type=f32)
  qa = jnp.dot(q, Wqd, preferred_element_type=f32) + ba1_ref[...]
  A1 = jax.nn.sigmoid(Z1.reshape(BB, T, 80) + qa[:, None, :])
  A1 = A1.reshape(BB * T, 80).astype(bf16)
  A2 = jax.nn.sigmoid(
      jnp.dot(A1, Wa2_ref[...].astype(bf16),
              preferred_element_type=f32) + ba2_ref[...]).astype(bf16)
  s = jnp.dot(A2, Wa3_ref[...].astype(bf16),
              preferred_element_type=f32) + ba3_ref[0, 0]
  s = s.reshape(BB, T) * (1.0 / jnp.sqrt(jnp.float32(D2)))
  pos = jax.lax.broadcasted_iota(jnp.int32, (BB, T), 1)
  s = jnp.where(pos < len_ref[...], s, jnp.float32(-(2.0 ** 32) + 1.0))
  s = s - jnp.max(s, axis=-1, keepdims=True)
  e = jnp.exp(s)
  w = e / jnp.sum(e, axis=-1, keepdims=True)
  rows = [jnp.dot(w[b:b + 1, :].astype(bf16), h3[b],
                  preferred_element_type=f32) for b in range(BB)]
  attn = jnp.concatenate(rows, axis=0)
  join_ref[...] = jnp.concatenate([utgc, attn], axis=-1)


def _fc_body(join_ref, gamma_ref, beta_ref, Wf1_ref, bf1_ref, a1_ref,
             Wf2_ref, bf2_ref, a2_ref, Wf3_ref, bf3_ref,
             out_ref, logit_ref):
  f32 = jnp.float32
  x = join_ref[...]
  mean = jnp.mean(x, axis=0, keepdims=True)
  var = jnp.mean((x - mean) ** 2, axis=0, keepdims=True)
  xn = (x - mean) / jnp.sqrt(var + 1e-3) * gamma_ref[...] + beta_ref[...]
  h = jnp.dot(xn, Wf1_ref[...], preferred_element_type=f32) + bf1_ref[...]
  h = jnp.maximum(h, 0.0)
  h = h + a1_ref[...] * jnp.minimum(h, 0.0)
  h2 = jnp.dot(h, Wf2_ref[...], preferred_element_type=f32) + bf2_ref[...]
  h2 = jnp.maximum(h2, 0.0)
  h2 = h2 + a2_ref[...] * jnp.minimum(h2, 0.0)
  logit = jnp.dot(h2, Wf3_ref[...], preferred_element_type=f32) + bf3_ref[...]
  m = jnp.max(logit, axis=-1, keepdims=True)
  e = jnp.exp(logit - m)
  out_ref[...] = e / jnp.sum(e, axis=-1, keepdims=True)
  logit_ref[...] = logit


def kernel(E_user, E_gender, E_item, E_cate, Wa1, ba1, Wa2, ba2, Wa3, ba3,
           gamma, beta, Wf1, bf1, alpha1, Wf2, bf2, alpha2, Wf3, bf3,
           user_id, gender, target_item_id, target_cate_id,
           hist_item_id, hist_cate_id, length):
  B, T = hist_item_id.shape
  D = E_item.shape[1]
  D2 = 2 * D
  f32 = jnp.float32
  i32 = jnp.int32

  Ei_p, Ec_p, Eu_p, Eg_p = (_sublane_tiled(E_item), _sublane_tiled(E_cate),
                            _sublane_tiled(E_user), _sublane_tiled(E_gender))
  hi_idx = hist_item_id.reshape(B * T).astype(i32)
  hc_idx = hist_cate_id.reshape(B * T).astype(i32)
  u_idx = user_id.astype(i32)
  g_idx = gender.astype(i32)
  ti_idx = target_item_id.astype(i32)
  tc_idx = target_cate_id.astype(i32)

  BB = 64
  len_i = length.astype(i32).reshape(B, 1)

  NCH = 4
  Bc = B // NCH
  BTc = Bc * T
  attn_call = pl.pallas_call(
      functools.partial(_attn_body, BB=BB, T=T, D2=D2),
      grid=(Bc // BB,),
      in_specs=[
          pl.BlockSpec((BB * T, D), lambda i: (i, 0)),
          pl.BlockSpec((BB * T, D), lambda i: (i, 0)),
          pl.BlockSpec((BB, D), lambda i: (i, 0)),
          pl.BlockSpec((BB, D), lambda i: (i, 0)),
          pl.BlockSpec((BB, D), lambda i: (i, 0)),
          pl.BlockSpec((BB, D), lambda i: (i, 0)),
          pl.BlockSpec((BB, 1), lambda i: (i, 0)),
          pl.BlockSpec((4 * D2, 80), lambda i: (0, 0)),
          pl.BlockSpec((1, 80), lambda i: (0, 0)),
          pl.BlockSpec((80, 40), lambda i: (0, 0)),
          pl.BlockSpec((1, 40), lambda i: (0, 0)),
          pl.BlockSpec((40, 1), lambda i: (0, 0)),
          pl.BlockSpec((1, 1), lambda i: (0, 0)),
      ],
      out_specs=pl.BlockSpec((BB, 6 * D), lambda i: (i, 0)),
      out_shape=jax.ShapeDtypeStruct((Bc, 6 * D), f32),
  )
  joins = []
  for k in range(NCH):
    st, sb = k * BTc, k * Bc
    hi, hc, u, g, ti, tc = _sc_gather(
        Ei_p, Ec_p, Eu_p, Eg_p,
        hi_idx[st:st + BTc], hc_idx[st:st + BTc],
        u_idx[sb:sb + Bc], g_idx[sb:sb + Bc],
        ti_idx[sb:sb + Bc], tc_idx[sb:sb + Bc])
    joins.append(attn_call(
        hi, hc, u, g, ti, tc, len_i[sb:sb + Bc], Wa1, ba1.reshape(1, -1),
        Wa2, ba2.reshape(1, -1), Wa3, ba3.reshape(1, 1)))
  join = jnp.concatenate(joins, axis=0)

  out, logit = pl.pallas_call(
      _fc_body,
      out_shape=(jax.ShapeDtypeStruct((B, 2), f32),
                 jax.ShapeDtypeStruct((B, 2), f32)),
  )(join, gamma.reshape(1, -1), beta.reshape(1, -1),
    Wf1, bf1.reshape(1, -1), alpha1.reshape(1, -1),
    Wf2, bf2.reshape(1, -1), alpha2.reshape(1, -1),
    Wf3, bf3.reshape(1, -1))
  return out, logit
